# baseline (speedup 1.0000x reference)
import functools

import jax
import jax.numpy as jnp
from jax import lax
from jax.experimental import pallas as pl
from jax.experimental.pallas import tpu as pltpu
from jax.experimental.pallas import tpu_sc as plsc

WS = 9
KK = 8
H = 256
W = 256
T = 4
C = 3
NW = 32
QG = 16
THIRD = 86
NP = 3
OCH = 4

_SORT9 = ((0, 3), (1, 7), (2, 5), (4, 8),
          (0, 7), (2, 4), (3, 8), (5, 6),
          (0, 2), (1, 3), (4, 5), (7, 8),
          (1, 4), (3, 6), (5, 7),
          (0, 1), (2, 4), (3, 5), (6, 8),
          (2, 3), (4, 5), (6, 7),
          (1, 2), (3, 4), (5, 6))
_BITONIC8 = ((0, 4), (1, 5), (2, 6), (3, 7),
             (0, 2), (1, 3), (4, 6), (5, 7),
             (0, 1), (2, 3), (4, 5), (6, 7))


@functools.partial(jax.jit, static_argnames=("dt", "row0", "nrows"))
def _sc_align_t(f0t, nbt, bit, bjt, dt, row0, nrows):
    qrows = nrows // NW
    ng = qrows * W // QG
    ogrp = ng // OCH
    owords = (qrows // OCH) * W * KK * 3
    mesh = plsc.VectorSubcoreMesh(core_axis_name="c", subcore_axis_name="s")

    @functools.partial(
        pl.kernel,
        out_type=jax.ShapeDtypeStruct((nrows * W * KK * 3,), jnp.float32),
        mesh=mesh,
        compiler_params=pltpu.CompilerParams(needs_layout_passes=False),
        scratch_types=[
            pltpu.VMEM((THIRD * W,), jnp.float32),
            pltpu.VMEM((THIRD * W,), jnp.float32),
            pltpu.VMEM((THIRD * W,), jnp.float32),
            pltpu.VMEM((qrows * W,), jnp.float32),
            pltpu.VMEM((qrows * W,), jnp.float32),
            pltpu.VMEM((qrows * W,), jnp.float32),
            pltpu.VMEM((qrows * W,), jnp.int32),
            pltpu.VMEM((qrows * W,), jnp.int32),
            pltpu.VMEM((ng * KK * QG,), jnp.float32),
            pltpu.VMEM((ng * KK * QG,), jnp.int32),
            pltpu.VMEM((owords,), jnp.float32),
        ],
    )
    def k(f0_h, nb_h, bi_h, bj_h, out_h, fr0_v, fr1_v, fr2_v, q0_v,
          q1_v, q2_v, bi_v, bj_v, bd_v, bs_v, ob_v):
        cid = lax.axis_index("c")
        sid = lax.axis_index("s")
        wid = sid * 2 + cid
        qrow0 = row0 + wid * qrows
        inf_v = jnp.full((QG,), jnp.inf, jnp.float32)
        zero_i = jnp.zeros((QG,), jnp.int32)
        lanes = lax.iota(jnp.int32, QG)

        for c, qc_v in enumerate((q0_v, q1_v, q2_v)):
            pltpu.sync_copy(
                f0_h.at[pl.ds(c * (H * W) + qrow0 * W, qrows * W)], qc_v)
        pltpu.sync_copy(bi_h.at[pl.ds(qrow0 * W, qrows * W)], bi_v)
        pltpu.sync_copy(bj_h.at[pl.ds(qrow0 * W, qrows * W)], bj_v)

        def p_body(p, _1):
            lo = p * THIRD
            hi = jnp.where(p == NP - 1, H, lo + THIRD)
            r0 = jnp.where(p == NP - 1, H - THIRD, lo)
            for c, fc_v in enumerate((fr0_v, fr1_v, fr2_v)):
                pltpu.sync_copy(
                    nb_h.at[pl.ds(c * (H * W) + r0 * W, THIRD * W)], fc_v)

            def g_body(g, _2):
                qb = g * QG
                biv0 = bi_v[pl.ds(qb, QG)]
                tv = jnp.clip(biv0 + (WS // 2), 0, H - 1)
                bv = jnp.clip(biv0 - (WS // 2), 0, H - 1)
                nact = plsc.all_reduce_population_count(
                    (tv >= lo) & (bv < hi))
                nbefore = plsc.all_reduce_population_count(bv < lo)

                @pl.when(nact[0] != 0)
                def _run():
                    first = nbefore[0] == 0
                    biv = bi_v[pl.ds(qb, QG)]
                    bjv = bj_v[pl.ds(qb, QG)]
                    q0 = q0_v[pl.ds(qb, QG)]
                    q1 = q1_v[pl.ds(qb, QG)]
                    q2 = q2_v[pl.ds(qb, QG)]

                    bd = tuple(
                        jnp.where(first, inf_v,
                                  bd_v[pl.ds((g * KK + kq) * QG, QG)])
                        for kq in range(KK))
                    bs = tuple(
                        jnp.where(first, zero_i,
                                  bs_v[pl.ds((g * KK + kq) * QG, QG)])
                        for kq in range(KK))

                    def oi_body(ii, car):
                        cbd, cbs = car
                        cbd = list(cbd)
                        cbs = list(cbs)
                        ci = jnp.clip(biv + (ii - WS // 2), 0, H - 1)
                        inr = (ci >= lo) & (ci < hi)
                        cb = jnp.where(inr, ci - r0, 0) << 8
                        nd = []
                        ns = []
                        for oj in range(WS):
                            cj = jnp.clip(bjv + (oj - WS // 2), 0, W - 1)
                            idx = cb + cj
                            g0 = plsc.load_gather(fr0_v, [idx])
                            g1 = plsc.load_gather(fr1_v, [idx])
                            g2 = plsc.load_gather(fr2_v, [idx])
                            s0 = q0 - g0
                            s1 = q1 - g1
                            s2 = q2 - g2
                            d = s0 * s0 + s1 * s1
                            d = d + s2 * s2
                            nd.append(jnp.where(inr, d, jnp.inf))
                            ns.append(zero_i + (ii * WS + oj))
                        for (i, j) in _SORT9:
                            m = nd[j] < nd[i]
                            lo_d = jnp.where(m, nd[j], nd[i])
                            hi_d = jnp.where(m, nd[i], nd[j])
                            lo_s = jnp.where(m, ns[j], ns[i])
                            hi_s = jnp.where(m, ns[i], ns[j])
                            nd[i], nd[j] = lo_d, hi_d
                            ns[i], ns[j] = lo_s, hi_s
                        for i in range(KK):
                            m = nd[7 - i] < cbd[i]
                            cbd[i] = jnp.where(m, nd[7 - i], cbd[i])
                            cbs[i] = jnp.where(m, ns[7 - i], cbs[i])
                        for (i, j) in _BITONIC8:
                            m = cbd[j] < cbd[i]
                            lo_d = jnp.where(m, cbd[j], cbd[i])
                            hi_d = jnp.where(m, cbd[i], cbd[j])
                            lo_s = jnp.where(m, cbs[j], cbs[i])
                            hi_s = jnp.where(m, cbs[i], cbs[j])
                            cbd[i], cbd[j] = lo_d, hi_d
                            cbs[i], cbs[j] = lo_s, hi_s
                        return (tuple(cbd), tuple(cbs))

                    bd, bs = lax.fori_loop(0, WS, oi_body, (bd, bs))
                    for kq in range(KK):
                        bd_v[pl.ds((g * KK + kq) * QG, QG)] = bd[kq]
                        bs_v[pl.ds((g * KK + kq) * QG, QG)] = bs[kq]

                return 0

            lax.fori_loop(0, ng, g_body, 0)
            return 0

        lax.fori_loop(0, NP, p_body, 0)

        dt_v = jnp.full((QG,), float(dt), jnp.float32)
        lane3k = lanes * (KK * 3)

        def chunk_body(cu, _1):
            def og_body(gg, _2):
                g = cu * ogrp + gg
                qb = g * QG
                biv = bi_v[pl.ds(qb, QG)]
                bjv = bj_v[pl.ds(qb, QG)]
                irow = qrow0 + (g >> 4)
                jv = ((g & 15) * QG) + lanes
                base = gg * QG * (KK * 3)
                for kq in range(KK):
                    s = bs_v[pl.ds((g * KK + kq) * QG, QG)]
                    qoi = (s * 7282) >> 16
                    ojj = s - qoi * WS
                    ci = jnp.clip(biv + (qoi - WS // 2), 0, H - 1)
                    cj = jnp.clip(bjv + (ojj - WS // 2), 0, W - 1)
                    dh = (ci - irow).astype(jnp.float32)
                    dw = (cj - jv).astype(jnp.float32)
                    idx0 = lane3k + (base + kq * 3)
                    plsc.store_scatter(ob_v, [idx0], dt_v)
                    plsc.store_scatter(ob_v, [idx0 + 1], dw)
                    plsc.store_scatter(ob_v, [idx0 + 2], dh)
                return 0

            lax.fori_loop(0, ogrp, og_body, 0)
            off = (((qrow0 - row0) + cu * (qrows // OCH)) * W) * (KK * 3)
            pltpu.sync_copy(ob_v, out_h.at[pl.ds(off, owords)])
            return 0

        lax.fori_loop(0, OCH, chunk_body, 0)

    return k(f0t, nbt, bit, bjt)


def kernel(vid, flows):
    assert vid.shape == (1, T, C, H, W), vid.shape
    f0 = vid[0].reshape(T, C * H * W)
    nb = jnp.stack([vid[0, 1], vid[0, 2], vid[0, 3], vid[0, 2]])
    nb = nb.reshape(T, C * H * W)
    fl = flows[0, :, 0]
    gi = jnp.arange(H, dtype=fl.dtype)[None, :, None]
    gj = jnp.arange(W, dtype=fl.dtype)[None, None, :]
    bi = jnp.clip(gi + jnp.round(fl[:, 0]), 0, H - 1).astype(jnp.int32)
    bj = jnp.clip(gj + jnp.round(fl[:, 1]), 0, W - 1).astype(jnp.int32)
    bi = bi.reshape(T, H * W)
    bj = bj.reshape(T, H * W)
    tparts = []
    for t in range(T):
        dt = 1 if t < T - 1 else -1
        if t in (0, T - 1):
            halves = []
            for row0 in (0, H // 2):
                pf = _sc_align_t(f0[t], nb[t], bi[t], bj[t], dt=dt,
                                 row0=row0, nrows=H // 2)
                halves.append(pf.reshape(1, 1, 1, H // 2, W, KK, 3))
            tparts.append(jnp.concatenate(halves, axis=3))
        else:
            pf = _sc_align_t(f0[t], nb[t], bi[t], bj[t], dt=dt,
                             row0=0, nrows=H)
            tparts.append(pf.reshape(1, 1, 1, H, W, KK, 3))
    return jnp.concatenate(tparts, axis=2)


# speedup vs baseline: 1.1994x; 1.1994x over previous
import functools

import jax
import jax.numpy as jnp
from jax import lax
from jax.experimental import pallas as pl
from jax.experimental.pallas import tpu as pltpu
from jax.experimental.pallas import tpu_sc as plsc

WS = 9
KK = 8
H = 256
W = 256
T = 4
C = 3
NW = 32
QG = 16
THIRD = 86
NP = 3
OCH = 4

_SORT9 = ((0, 3), (1, 7), (2, 5), (4, 8),
          (0, 7), (2, 4), (3, 8), (5, 6),
          (0, 2), (1, 3), (4, 5), (7, 8),
          (1, 4), (3, 6), (5, 7),
          (0, 1), (2, 4), (3, 5), (6, 8),
          (2, 3), (4, 5), (6, 7),
          (1, 2), (3, 4), (5, 6))
_BITONIC8 = ((0, 4), (1, 5), (2, 6), (3, 7),
             (0, 2), (1, 3), (4, 6), (5, 7),
             (0, 1), (2, 3), (4, 5), (6, 7))


@functools.partial(jax.jit, static_argnames=("dt", "row0", "nrows"))
def _sc_align_t(f0t, nbt, bit, bjt, dt, row0, nrows):
    qrows = nrows // NW
    ng = qrows * W // QG
    ogrp = ng // OCH
    owords = (qrows // OCH) * W * KK * 3
    mesh = plsc.VectorSubcoreMesh(core_axis_name="c", subcore_axis_name="s")

    @functools.partial(
        pl.kernel,
        out_type=jax.ShapeDtypeStruct((nrows * W * KK * 3,), jnp.float32),
        mesh=mesh,
        compiler_params=pltpu.CompilerParams(needs_layout_passes=False),
        scratch_types=[
            pltpu.VMEM((THIRD * W,), jnp.float32),
            pltpu.VMEM((THIRD * W,), jnp.float32),
            pltpu.VMEM((THIRD * W,), jnp.float32),
            pltpu.VMEM((qrows * W,), jnp.float32),
            pltpu.VMEM((qrows * W,), jnp.float32),
            pltpu.VMEM((qrows * W,), jnp.float32),
            pltpu.VMEM((qrows * W,), jnp.int32),
            pltpu.VMEM((qrows * W,), jnp.int32),
            pltpu.VMEM((ng * KK * QG,), jnp.float32),
            pltpu.VMEM((ng * KK * QG,), jnp.int32),
            pltpu.VMEM((owords,), jnp.float32),
        ],
    )
    def k(f0_h, nb_h, bi_h, bj_h, out_h, fr0_v, fr1_v, fr2_v, q0_v,
          q1_v, q2_v, bi_v, bj_v, bd_v, bs_v, ob_v):
        cid = lax.axis_index("c")
        sid = lax.axis_index("s")
        wid = sid * 2 + cid
        qrow0 = row0 + wid * qrows
        inf_v = jnp.full((QG,), jnp.inf, jnp.float32)
        zero_i = jnp.zeros((QG,), jnp.int32)
        lanes = lax.iota(jnp.int32, QG)

        for c, qc_v in enumerate((q0_v, q1_v, q2_v)):
            pltpu.sync_copy(
                f0_h.at[pl.ds(c * (H * W) + qrow0 * W, qrows * W)], qc_v)
        pltpu.sync_copy(bi_h.at[pl.ds(qrow0 * W, qrows * W)], bi_v)
        pltpu.sync_copy(bj_h.at[pl.ds(qrow0 * W, qrows * W)], bj_v)

        def p_body(p, _1):
            lo = p * THIRD
            hi = jnp.where(p == NP - 1, H, lo + THIRD)
            r0 = jnp.where(p == NP - 1, H - THIRD, lo)
            for c, fc_v in enumerate((fr0_v, fr1_v, fr2_v)):
                pltpu.sync_copy(
                    nb_h.at[pl.ds(c * (H * W) + r0 * W, THIRD * W)], fc_v)

            def g_body(g, _2):
                qb = g * QG
                biv0 = bi_v[pl.ds(qb, QG)]
                tv = jnp.clip(biv0 + (WS // 2), 0, H - 1)
                bv = jnp.clip(biv0 - (WS // 2), 0, H - 1)
                nact = plsc.all_reduce_population_count(
                    (tv >= lo) & (bv < hi))
                nbefore = plsc.all_reduce_population_count(bv < lo)

                @pl.when(nact[0] != 0)
                def _run():
                    first = nbefore[0] == 0
                    biv = bi_v[pl.ds(qb, QG)]
                    bjv = bj_v[pl.ds(qb, QG)]
                    q0 = q0_v[pl.ds(qb, QG)]
                    q1 = q1_v[pl.ds(qb, QG)]
                    q2 = q2_v[pl.ds(qb, QG)]

                    bd = tuple(
                        jnp.where(first, inf_v,
                                  bd_v[pl.ds((g * KK + kq) * QG, QG)])
                        for kq in range(KK))
                    bs = tuple(
                        jnp.where(first, zero_i,
                                  bs_v[pl.ds((g * KK + kq) * QG, QG)])
                        for kq in range(KK))

                    def oi_body(ii, car):
                        cbd, cbs = car
                        cbd = list(cbd)
                        cbs = list(cbs)
                        ci = jnp.clip(biv + (ii - WS // 2), 0, H - 1)
                        inr = (ci >= lo) & (ci < hi)
                        cb = jnp.where(inr, ci - r0, 0) << 8
                        nd = []
                        ns = []
                        for oj in range(WS):
                            cj = jnp.clip(bjv + (oj - WS // 2), 0, W - 1)
                            idx = cb + cj
                            g0 = plsc.load_gather(fr0_v, [idx])
                            g1 = plsc.load_gather(fr1_v, [idx])
                            g2 = plsc.load_gather(fr2_v, [idx])
                            s0 = q0 - g0
                            s1 = q1 - g1
                            s2 = q2 - g2
                            d = s0 * s0 + s1 * s1
                            d = d + s2 * s2
                            nd.append(jnp.where(inr, d, jnp.inf))
                            ns.append(zero_i + (ii * WS + oj))
                        for (i, j) in _SORT9:
                            m = nd[j] < nd[i]
                            lo_d = jnp.where(m, nd[j], nd[i])
                            hi_d = jnp.where(m, nd[i], nd[j])
                            lo_s = jnp.where(m, ns[j], ns[i])
                            hi_s = jnp.where(m, ns[i], ns[j])
                            nd[i], nd[j] = lo_d, hi_d
                            ns[i], ns[j] = lo_s, hi_s
                        for i in range(KK):
                            m = nd[7 - i] < cbd[i]
                            cbd[i] = jnp.where(m, nd[7 - i], cbd[i])
                            cbs[i] = jnp.where(m, ns[7 - i], cbs[i])
                        for (i, j) in _BITONIC8:
                            m = cbd[j] < cbd[i]
                            lo_d = jnp.where(m, cbd[j], cbd[i])
                            hi_d = jnp.where(m, cbd[i], cbd[j])
                            lo_s = jnp.where(m, cbs[j], cbs[i])
                            hi_s = jnp.where(m, cbs[i], cbs[j])
                            cbd[i], cbd[j] = lo_d, hi_d
                            cbs[i], cbs[j] = lo_s, hi_s
                        return (tuple(cbd), tuple(cbs))

                    bd, bs = lax.fori_loop(0, WS, oi_body, (bd, bs))
                    for kq in range(KK):
                        bd_v[pl.ds((g * KK + kq) * QG, QG)] = bd[kq]
                        bs_v[pl.ds((g * KK + kq) * QG, QG)] = bs[kq]

                return 0

            lax.fori_loop(0, ng, g_body, 0)
            return 0

        lax.fori_loop(0, NP, p_body, 0)

        dt_v = jnp.full((QG,), float(dt), jnp.float32)
        lane3k = lanes * (KK * 3)

        def chunk_body(cu, _1):
            def og_body(gg, _2):
                g = cu * ogrp + gg
                qb = g * QG
                biv = bi_v[pl.ds(qb, QG)]
                bjv = bj_v[pl.ds(qb, QG)]
                irow = qrow0 + (g >> 4)
                jv = ((g & 15) * QG) + lanes
                base = gg * QG * (KK * 3)
                for kq in range(KK):
                    s = bs_v[pl.ds((g * KK + kq) * QG, QG)]
                    qoi = (s * 7282) >> 16
                    ojj = s - qoi * WS
                    ci = jnp.clip(biv + (qoi - WS // 2), 0, H - 1)
                    cj = jnp.clip(bjv + (ojj - WS // 2), 0, W - 1)
                    dh = (ci - irow).astype(jnp.float32)
                    dw = (cj - jv).astype(jnp.float32)
                    idx0 = lane3k + (base + kq * 3)
                    plsc.store_scatter(ob_v, [idx0], dt_v)
                    plsc.store_scatter(ob_v, [idx0 + 1], dw)
                    plsc.store_scatter(ob_v, [idx0 + 2], dh)
                return 0

            lax.fori_loop(0, ogrp, og_body, 0)
            off = (((qrow0 - row0) + cu * (qrows // OCH)) * W) * (KK * 3)
            pltpu.sync_copy(ob_v, out_h.at[pl.ds(off, owords)])
            return 0

        lax.fori_loop(0, OCH, chunk_body, 0)

    return k(f0t, nbt, bit, bjt)


def kernel(vid, flows):
    assert vid.shape == (1, T, C, H, W), vid.shape
    f0 = vid[0].reshape(T, C * H * W)
    nb = jnp.stack([vid[0, 1], vid[0, 2], vid[0, 3], vid[0, 2]])
    nb = nb.reshape(T, C * H * W)
    fl = flows[0, :, 0]
    gi = jnp.arange(H, dtype=fl.dtype)[None, :, None]
    gj = jnp.arange(W, dtype=fl.dtype)[None, None, :]
    bi = jnp.clip(gi + jnp.round(fl[:, 0]), 0, H - 1).astype(jnp.int32)
    bj = jnp.clip(gj + jnp.round(fl[:, 1]), 0, W - 1).astype(jnp.int32)
    bi = bi.reshape(T, H * W)
    bj = bj.reshape(T, H * W)
    tparts = []
    for t in range(T):
        dt = 1 if t < T - 1 else -1
        pf = _sc_align_t(f0[t], nb[t], bi[t], bj[t], dt=dt, row0=0, nrows=H)
        tparts.append(pf.reshape(1, 1, 1, H, W, KK, 3))
    return jnp.concatenate(tparts, axis=2)
